# baseline (speedup 1.0000x reference)
import functools

import jax
import jax.numpy as jnp
from jax import lax
from jax.experimental import pallas as pl
from jax.experimental.pallas import tpu as pltpu
from jax.experimental.pallas import tpu_sc as plsc

N_NODES = 100000
N_LINKS = 6400000
LPN = 64

NC = 2
NS = 16
NW = NC * NS

LINKS_PER_W = N_LINKS // NW
C1 = 1600
NCH1 = LINKS_PER_W // C1
VEC1 = C1 // 16
NBUF1 = 4

B2 = 160
NCH2 = N_NODES // B2
FULL_W = NCH2 % NW
BASE_K = NCH2 // NW


def _worker_id():
    return lax.axis_index("c") * NS + lax.axis_index("s")


def _phase1_body(array_hbm, len_hbm, head_hbm, tail_hbm, grad_hbm,
                 array_v, h0, h1, h2, h3, t0, t1, t2, t3,
                 l0, l1, l2, l3, g0, g1, g2, g3,
                 asem, ls0, ls1, ls2, ls3, ss0, ss1, ss2, ss3):
    wid = _worker_id()
    base = wid * LINKS_PER_W

    ARR_SEC = 5
    SECW = N_NODES // ARR_SEC
    for i in range(ARR_SEC):
        s = lax.rem(wid + i, ARR_SEC)
        sl = pl.ds(s * SECW, SECW)
        pltpu.async_copy(array_hbm.at[sl], array_v.at[sl], asem)

    bufs = ((h0, t0, l0, g0, ls0, ss0),
            (h1, t1, l1, g1, ls1, ss1),
            (h2, t2, l2, g2, ls2, ss2),
            (h3, t3, l3, g3, ls3, ss3))

    def issue_loads(j, b):
        h, t, ln, _, sem, _ = bufs[b]
        off = base + j * C1
        pltpu.async_copy(head_hbm.at[pl.ds(off, C1)], h, sem)
        pltpu.async_copy(tail_hbm.at[pl.ds(off, C1)], t, sem)
        pltpu.async_copy(len_hbm.at[pl.ds(off, C1)], ln, sem)

    def wait_loads(b):
        h, t, ln, _, sem, _ = bufs[b]
        pltpu.make_async_copy(head_hbm.at[pl.ds(0, C1)], h, sem).wait()
        pltpu.make_async_copy(tail_hbm.at[pl.ds(0, C1)], t, sem).wait()
        pltpu.make_async_copy(len_hbm.at[pl.ds(0, C1)], ln, sem).wait()

    def compute(b):
        h, t, ln, g, _, _ = bufs[b]

        def inner(i, _):
            for u in range(5):
                sl = pl.ds((i * 5 + u) * 16, 16)
                ah = plsc.load_gather(array_v, [h[sl]])
                at = plsc.load_gather(array_v, [t[sl]])
                g[sl] = (ah - at) / ln[sl]
            return 0

        lax.fori_loop(0, VEC1 // 5, inner, 0)

    def issue_store(j, b):
        _, _, _, g, _, sem = bufs[b]
        pltpu.async_copy(g, grad_hbm.at[pl.ds(base + j * C1, C1)], sem)

    def wait_store(b):
        _, _, _, g, _, sem = bufs[b]
        pltpu.make_async_copy(g, grad_hbm.at[pl.ds(0, C1)], sem).wait()

    for u in range(NBUF1):
        issue_loads(u, u)
    for i in range(ARR_SEC):
        asl = pl.ds(0, SECW)
        pltpu.make_async_copy(array_hbm.at[asl], array_v.at[asl], asem).wait()

    def outer(gi, _):
        for u in range(NBUF1):
            j = gi * NBUF1 + u
            wait_loads(u)

            @pl.when(j >= NBUF1)
            def _():
                wait_store(u)

            compute(u)
            issue_store(j, u)

            @pl.when(j + NBUF1 < NCH1)
            def _():
                issue_loads(j + NBUF1, u)
        return 0

    lax.fori_loop(0, NCH1 // NBUF1, outer, 0)
    wait_loads(0)
    wait_store(0)
    compute(0)
    issue_store(NCH1 - 1, 0)
    for u in range(NBUF1):
        wait_store(u)


def _phase2_body(grad_hbm, links_hbm, out_hbm,
                 links0, links1, vals0, vals1, out0, out1,
                 ls0, ls1, gs0, gs1, os0, os1):
    wid = _worker_id()
    extra = wid < FULL_W
    lanes = lax.iota(jnp.int32, 16)
    MAXK = BASE_K + 1

    bufs = ((links0, vals0, out0, ls0, gs0, os0),
            (links1, vals1, out1, ls1, gs1, os1))

    def chunk(k):
        return wid + k * NW

    def issue_links(k, b):
        l, _, _, ls, _, _ = bufs[b]
        pltpu.async_copy(
            links_hbm.at[pl.ds(chunk(k) * B2 * LPN, B2 * LPN)], l, ls)

    def wait_links(b):
        l, _, _, ls, _, _ = bufs[b]
        pltpu.make_async_copy(
            links_hbm.at[pl.ds(0, B2 * LPN)], l, ls).wait()

    def issue_gather(b):
        l, v, _, _, gs, _ = bufs[b]
        pltpu.async_copy(grad_hbm.at[l], v, gs)

    def wait_gather(b):
        l, v, _, _, gs, _ = bufs[b]
        pltpu.make_async_copy(grad_hbm.at[l], v, gs).wait()

    def reduce(b):
        _, v, o, _, _, _ = bufs[b]

        def group(g, _):
            base = (g * 16 + lanes) * LPN

            def slot8(s8, acc):
                for u in range(8):
                    acc = acc + plsc.load_gather(v, [base + (s8 * 8 + u)])
                return acc

            acc = lax.fori_loop(0, LPN // 8, slot8,
                                jnp.zeros((16,), jnp.float32))
            o[pl.ds(g * 16, 16)] = acc * (1.0 / LPN)
            return 0

        lax.fori_loop(0, B2 // 16, group, 0)

    def issue_out(k, b):
        _, _, o, _, _, os = bufs[b]
        pltpu.async_copy(o, out_hbm.at[pl.ds(chunk(k) * B2, B2)], os)

    def wait_out(b):
        _, _, o, _, _, os = bufs[b]
        pltpu.make_async_copy(o, out_hbm.at[pl.ds(0, B2)], os).wait()

    issue_links(0, 0)
    issue_links(1, 1)
    for k in range(MAXK + 1):
        b = k % 2
        ob = 1 - b
        if k < MAXK - 1:
            wait_links(b)
            issue_gather(b)
        elif k == MAXK - 1:
            @pl.when(extra)
            def _():
                wait_links(b)
                issue_gather(b)
        if k >= 1:
            def stage(ob=ob, k=k):
                wait_gather(ob)
                if k + 1 < MAXK - 1:
                    issue_links(k + 1, ob)
                elif k + 1 == MAXK - 1:
                    @pl.when(extra)
                    def _():
                        issue_links(k + 1, ob)
                if k >= 3:
                    wait_out(ob)
                reduce(ob)
                issue_out(k - 1, ob)

            if k - 1 < MAXK - 1:
                stage()
            else:
                @pl.when(extra)
                def _():
                    stage()
    wait_out(0)
    wait_out(1)


_mesh = plsc.VectorSubcoreMesh(core_axis_name="c", subcore_axis_name="s")
_cparams = pltpu.CompilerParams(needs_layout_passes=False)

_phase1 = functools.partial(
    pl.kernel,
    out_type=jax.ShapeDtypeStruct((N_LINKS,), jnp.float32),
    mesh=_mesh,
    compiler_params=_cparams,
    scratch_types=(
        [pltpu.VMEM((N_NODES,), jnp.float32)]
        + [pltpu.VMEM((C1,), jnp.int32)] * 8
        + [pltpu.VMEM((C1,), jnp.float32)] * 8
        + [pltpu.SemaphoreType.DMA] * 9
    ),
)(_phase1_body)

_phase2 = functools.partial(
    pl.kernel,
    out_type=jax.ShapeDtypeStruct((N_NODES,), jnp.float32),
    mesh=_mesh,
    compiler_params=_cparams,
    scratch_types=[
        pltpu.VMEM((B2 * LPN,), jnp.int32),
        pltpu.VMEM((B2 * LPN,), jnp.int32),
        pltpu.VMEM((B2 * LPN,), jnp.float32),
        pltpu.VMEM((B2 * LPN,), jnp.float32),
        pltpu.VMEM((B2,), jnp.float32),
        pltpu.VMEM((B2,), jnp.float32),
        pltpu.SemaphoreType.DMA,
        pltpu.SemaphoreType.DMA,
        pltpu.SemaphoreType.DMA,
        pltpu.SemaphoreType.DMA,
        pltpu.SemaphoreType.DMA,
        pltpu.SemaphoreType.DMA,
    ],
)(_phase2_body)


def kernel(array, length_of_link, node_at_link_head, node_at_link_tail,
           links_at_node):
    grad = _phase1(array, length_of_link, node_at_link_head,
                   node_at_link_tail)
    return _phase2(grad, links_at_node.reshape(N_NODES * LPN))


# speedup vs baseline: 1.0037x; 1.0037x over previous
import functools

import jax
import jax.numpy as jnp
from jax import lax
from jax.experimental import pallas as pl
from jax.experimental.pallas import tpu as pltpu
from jax.experimental.pallas import tpu_sc as plsc

N_NODES = 100000
N_LINKS = 6400000
LPN = 64

NC = 2
NS = 16
NW = NC * NS

LINKS_PER_W = N_LINKS // NW
C1 = 2000
NCH1 = LINKS_PER_W // C1
VEC1 = C1 // 16
NBUF1 = 3

B2 = 160
NCH2 = N_NODES // B2
FULL_W = NCH2 % NW
BASE_K = NCH2 // NW


def _worker_id():
    return lax.axis_index("c") * NS + lax.axis_index("s")


def _phase1_body(array_hbm, len_hbm, head_hbm, tail_hbm, grad_hbm,
                 array_v, h0, h1, h2, t0, t1, t2,
                 l0, l1, l2, g0, g1, g2,
                 asem, ls0, ls1, ls2, ss0, ss1, ss2):
    wid = _worker_id()
    base = wid * LINKS_PER_W

    ARR_SEC = 5
    SECW = N_NODES // ARR_SEC
    for i in range(ARR_SEC):
        s = lax.rem(wid + i, ARR_SEC)
        sl = pl.ds(s * SECW, SECW)
        pltpu.async_copy(array_hbm.at[sl], array_v.at[sl], asem)

    bufs = ((h0, t0, l0, g0, ls0, ss0),
            (h1, t1, l1, g1, ls1, ss1),
            (h2, t2, l2, g2, ls2, ss2))

    def issue_loads(j, b):
        h, t, ln, _, sem, _ = bufs[b]
        off = base + j * C1
        pltpu.async_copy(head_hbm.at[pl.ds(off, C1)], h, sem)
        pltpu.async_copy(tail_hbm.at[pl.ds(off, C1)], t, sem)
        pltpu.async_copy(len_hbm.at[pl.ds(off, C1)], ln, sem)

    def wait_loads(b):
        h, t, ln, _, sem, _ = bufs[b]
        pltpu.make_async_copy(head_hbm.at[pl.ds(0, C1)], h, sem).wait()
        pltpu.make_async_copy(tail_hbm.at[pl.ds(0, C1)], t, sem).wait()
        pltpu.make_async_copy(len_hbm.at[pl.ds(0, C1)], ln, sem).wait()

    def compute(b):
        h, t, ln, g, _, _ = bufs[b]

        def inner(i, _):
            for u in range(5):
                sl = pl.ds((i * 5 + u) * 16, 16)
                ah = plsc.load_gather(array_v, [h[sl]])
                at = plsc.load_gather(array_v, [t[sl]])
                g[sl] = (ah - at) / ln[sl]
            return 0

        lax.fori_loop(0, VEC1 // 5, inner, 0)

    def issue_store(j, b):
        _, _, _, g, _, sem = bufs[b]
        pltpu.async_copy(g, grad_hbm.at[pl.ds(base + j * C1, C1)], sem)

    def wait_store(b):
        _, _, _, g, _, sem = bufs[b]
        pltpu.make_async_copy(g, grad_hbm.at[pl.ds(0, C1)], sem).wait()

    for u in range(NBUF1):
        issue_loads(u, u)
    for i in range(ARR_SEC):
        asl = pl.ds(0, SECW)
        pltpu.make_async_copy(array_hbm.at[asl], array_v.at[asl], asem).wait()

    def outer(gi, _):
        for u in range(NBUF1):
            j = gi * NBUF1 + u
            wait_loads(u)

            @pl.when(j >= NBUF1)
            def _():
                wait_store(u)

            compute(u)
            issue_store(j, u)

            @pl.when(j + NBUF1 < NCH1)
            def _():
                issue_loads(j + NBUF1, u)
        return 0

    lax.fori_loop(0, NCH1 // NBUF1, outer, 0)
    wait_loads(0)
    wait_store(0)
    compute(0)
    issue_store(NCH1 - 1, 0)
    for u in range(NBUF1):
        wait_store(u)


def _phase2_body(grad_hbm, links_hbm, out_hbm,
                 links0, links1, vals0, vals1, out0, out1,
                 ls0, ls1, gs0, gs1, os0, os1):
    wid = _worker_id()
    extra = wid < FULL_W
    lanes = lax.iota(jnp.int32, 16)
    MAXK = BASE_K + 1

    bufs = ((links0, vals0, out0, ls0, gs0, os0),
            (links1, vals1, out1, ls1, gs1, os1))

    def chunk(k):
        return wid + k * NW

    def issue_links(k, b):
        l, _, _, ls, _, _ = bufs[b]
        pltpu.async_copy(
            links_hbm.at[pl.ds(chunk(k) * B2 * LPN, B2 * LPN)], l, ls)

    def wait_links(b):
        l, _, _, ls, _, _ = bufs[b]
        pltpu.make_async_copy(
            links_hbm.at[pl.ds(0, B2 * LPN)], l, ls).wait()

    def issue_gather(b):
        l, v, _, _, gs, _ = bufs[b]
        pltpu.async_copy(grad_hbm.at[l], v, gs)

    def wait_gather(b):
        l, v, _, _, gs, _ = bufs[b]
        pltpu.make_async_copy(grad_hbm.at[l], v, gs).wait()

    def reduce(b):
        _, v, o, _, _, _ = bufs[b]

        def group(g, _):
            base = (g * 16 + lanes) * LPN

            def slot8(s8, acc):
                for u in range(8):
                    acc = acc + plsc.load_gather(v, [base + (s8 * 8 + u)])
                return acc

            acc = lax.fori_loop(0, LPN // 8, slot8,
                                jnp.zeros((16,), jnp.float32))
            o[pl.ds(g * 16, 16)] = acc * (1.0 / LPN)
            return 0

        lax.fori_loop(0, B2 // 16, group, 0)

    def issue_out(k, b):
        _, _, o, _, _, os = bufs[b]
        pltpu.async_copy(o, out_hbm.at[pl.ds(chunk(k) * B2, B2)], os)

    def wait_out(b):
        _, _, o, _, _, os = bufs[b]
        pltpu.make_async_copy(o, out_hbm.at[pl.ds(0, B2)], os).wait()

    issue_links(0, 0)
    issue_links(1, 1)
    for k in range(MAXK + 1):
        b = k % 2
        ob = 1 - b
        if k < MAXK - 1:
            wait_links(b)
            issue_gather(b)
        elif k == MAXK - 1:
            @pl.when(extra)
            def _():
                wait_links(b)
                issue_gather(b)
        if k >= 1:
            def stage(ob=ob, k=k):
                wait_gather(ob)
                if k + 1 < MAXK - 1:
                    issue_links(k + 1, ob)
                elif k + 1 == MAXK - 1:
                    @pl.when(extra)
                    def _():
                        issue_links(k + 1, ob)
                if k >= 3:
                    wait_out(ob)
                reduce(ob)
                issue_out(k - 1, ob)

            if k - 1 < MAXK - 1:
                stage()
            else:
                @pl.when(extra)
                def _():
                    stage()
    wait_out(0)
    wait_out(1)


_mesh = plsc.VectorSubcoreMesh(core_axis_name="c", subcore_axis_name="s")
_cparams = pltpu.CompilerParams(needs_layout_passes=False)

_phase1 = functools.partial(
    pl.kernel,
    out_type=jax.ShapeDtypeStruct((N_LINKS,), jnp.float32),
    mesh=_mesh,
    compiler_params=_cparams,
    scratch_types=(
        [pltpu.VMEM((N_NODES,), jnp.float32)]
        + [pltpu.VMEM((C1,), jnp.int32)] * 6
        + [pltpu.VMEM((C1,), jnp.float32)] * 6
        + [pltpu.SemaphoreType.DMA] * 7
    ),
)(_phase1_body)

_phase2 = functools.partial(
    pl.kernel,
    out_type=jax.ShapeDtypeStruct((N_NODES,), jnp.float32),
    mesh=_mesh,
    compiler_params=_cparams,
    scratch_types=[
        pltpu.VMEM((B2 * LPN,), jnp.int32),
        pltpu.VMEM((B2 * LPN,), jnp.int32),
        pltpu.VMEM((B2 * LPN,), jnp.float32),
        pltpu.VMEM((B2 * LPN,), jnp.float32),
        pltpu.VMEM((B2,), jnp.float32),
        pltpu.VMEM((B2,), jnp.float32),
        pltpu.SemaphoreType.DMA,
        pltpu.SemaphoreType.DMA,
        pltpu.SemaphoreType.DMA,
        pltpu.SemaphoreType.DMA,
        pltpu.SemaphoreType.DMA,
        pltpu.SemaphoreType.DMA,
    ],
)(_phase2_body)


def kernel(array, length_of_link, node_at_link_head, node_at_link_tail,
           links_at_node):
    grad = _phase1(array, length_of_link, node_at_link_head,
                   node_at_link_tail)
    return _phase2(grad, links_at_node.reshape(N_NODES * LPN))


# speedup vs baseline: 1.0229x; 1.0192x over previous
import functools

import jax
import jax.numpy as jnp
from jax import lax
from jax.experimental import pallas as pl
from jax.experimental.pallas import tpu as pltpu
from jax.experimental.pallas import tpu_sc as plsc

N_NODES = 100000
N_LINKS = 6400000
LPN = 64

NC = 2
NS = 16
NW = NC * NS

LINKS_PER_W = N_LINKS // NW
C1 = 2000
NCH1 = LINKS_PER_W // C1
VEC1 = C1 // 16
NBUF1 = 3

B2 = 160
NCH2 = N_NODES // B2
FULL_W = NCH2 % NW
BASE_K = NCH2 // NW


def _worker_id():
    return lax.axis_index("c") * NS + lax.axis_index("s")


def _phase1_body(array_hbm, len_hbm, head_hbm, tail_hbm, grad_hbm,
                 array_v, h0, h1, h2, t0, t1, t2,
                 l0, l1, l2, g0, g1, g2,
                 asem, ls0, ls1, ls2, ss0, ss1, ss2):
    wid = _worker_id()
    base = wid * LINKS_PER_W

    ARR_SEC = 5
    SECW = N_NODES // ARR_SEC
    for i in range(ARR_SEC):
        s = lax.rem(wid + i, ARR_SEC)
        sl = pl.ds(s * SECW, SECW)
        pltpu.async_copy(array_hbm.at[sl], array_v.at[sl], asem)

    bufs = ((h0, t0, l0, g0, ls0, ss0),
            (h1, t1, l1, g1, ls1, ss1),
            (h2, t2, l2, g2, ls2, ss2))

    def issue_loads(j, b):
        h, t, ln, _, sem, _ = bufs[b]
        off = base + j * C1
        pltpu.async_copy(head_hbm.at[pl.ds(off, C1)], h, sem)
        pltpu.async_copy(tail_hbm.at[pl.ds(off, C1)], t, sem)
        pltpu.async_copy(len_hbm.at[pl.ds(off, C1)], ln, sem)

    def wait_loads(b):
        h, t, ln, _, sem, _ = bufs[b]
        pltpu.make_async_copy(head_hbm.at[pl.ds(0, C1)], h, sem).wait()
        pltpu.make_async_copy(tail_hbm.at[pl.ds(0, C1)], t, sem).wait()
        pltpu.make_async_copy(len_hbm.at[pl.ds(0, C1)], ln, sem).wait()

    def compute(b):
        h, t, ln, g, _, _ = bufs[b]

        def inner(i, _):
            for u in range(5):
                sl = pl.ds((i * 5 + u) * 16, 16)
                ah = plsc.load_gather(array_v, [h[sl]])
                at = plsc.load_gather(array_v, [t[sl]])
                g[sl] = (ah - at) / ln[sl]
            return 0

        lax.fori_loop(0, VEC1 // 5, inner, 0)

    def issue_store(j, b):
        _, _, _, g, _, sem = bufs[b]
        pltpu.async_copy(g, grad_hbm.at[pl.ds(base + j * C1, C1)], sem)

    def wait_store(b):
        _, _, _, g, _, sem = bufs[b]
        pltpu.make_async_copy(g, grad_hbm.at[pl.ds(0, C1)], sem).wait()

    for u in range(NBUF1):
        issue_loads(u, u)
    for i in range(ARR_SEC):
        asl = pl.ds(0, SECW)
        pltpu.make_async_copy(array_hbm.at[asl], array_v.at[asl], asem).wait()

    def outer(gi, _):
        for u in range(NBUF1):
            j = gi * NBUF1 + u
            wait_loads(u)

            @pl.when(j >= NBUF1)
            def _():
                wait_store(u)

            compute(u)
            issue_store(j, u)

            @pl.when(j + NBUF1 < NCH1)
            def _():
                issue_loads(j + NBUF1, u)
        return 0

    lax.fori_loop(0, NCH1 // NBUF1, outer, 0)
    wait_loads(0)
    wait_store(0)
    compute(0)
    issue_store(NCH1 - 1, 0)
    for u in range(NBUF1):
        wait_store(u)


def _phase2_body(grad_hbm, links_hbm, out_hbm,
                 l2d0, l2d1, links0, links1, vals0, vals1, out0, out1,
                 ls0, ls1, gs0, gs1, os0, os1):
    wid = _worker_id()
    extra = wid < FULL_W
    lanes = lax.iota(jnp.int32, 16)
    MAXK = BASE_K + 1

    bufs = ((l2d0, links0, vals0, out0, ls0, gs0, os0),
            (l2d1, links1, vals1, out1, ls1, gs1, os1))

    def chunk(k):
        return wid + k * NW

    def issue_links(k, b):
        l2, _, _, _, ls, _, _ = bufs[b]
        pltpu.async_copy(
            links_hbm.at[pl.ds(chunk(k) * B2, B2), :], l2, ls)

    def wait_links(b):
        l2, _, _, _, ls, _, _ = bufs[b]
        pltpu.make_async_copy(
            links_hbm.at[pl.ds(0, B2), :], l2, ls).wait()

    def flatten(b):
        l2, l1, _, _, _, _, _ = bufs[b]

        def row(r, _):
            for u in range(LPN // 16):
                l1[pl.ds(r * LPN + u * 16, 16)] = l2[r, pl.ds(u * 16, 16)]
            return 0

        lax.fori_loop(0, B2, row, 0)

    def issue_gather(b):
        _, l1, v, _, _, gs, _ = bufs[b]
        pltpu.async_copy(grad_hbm.at[l1], v, gs)

    def wait_gather(b):
        _, l1, v, _, _, gs, _ = bufs[b]
        pltpu.make_async_copy(grad_hbm.at[l1], v, gs).wait()

    def reduce(b):
        _, _, v, o, _, _, _ = bufs[b]

        def group(g, _):
            base = (g * 16 + lanes) * LPN

            def slot8(s8, acc):
                for u in range(8):
                    acc = acc + plsc.load_gather(v, [base + (s8 * 8 + u)])
                return acc

            acc = lax.fori_loop(0, LPN // 8, slot8,
                                jnp.zeros((16,), jnp.float32))
            o[pl.ds(g * 16, 16)] = acc * (1.0 / LPN)
            return 0

        lax.fori_loop(0, B2 // 16, group, 0)

    def issue_out(k, b):
        _, _, _, o, _, _, os = bufs[b]
        pltpu.async_copy(o, out_hbm.at[pl.ds(chunk(k) * B2, B2)], os)

    def wait_out(b):
        _, _, _, o, _, _, os = bufs[b]
        pltpu.make_async_copy(o, out_hbm.at[pl.ds(0, B2)], os).wait()

    issue_links(0, 0)
    issue_links(1, 1)
    for k in range(MAXK + 1):
        b = k % 2
        ob = 1 - b
        if k < MAXK - 1:
            wait_links(b)
            flatten(b)
            issue_gather(b)
        elif k == MAXK - 1:
            @pl.when(extra)
            def _():
                wait_links(b)
                flatten(b)
                issue_gather(b)
        if k >= 1:
            def stage(ob=ob, k=k):
                wait_gather(ob)
                if k + 1 < MAXK - 1:
                    issue_links(k + 1, ob)
                elif k + 1 == MAXK - 1:
                    @pl.when(extra)
                    def _():
                        issue_links(k + 1, ob)
                if k >= 3:
                    wait_out(ob)
                reduce(ob)
                issue_out(k - 1, ob)

            if k - 1 < MAXK - 1:
                stage()
            else:
                @pl.when(extra)
                def _():
                    stage()
    wait_out(0)
    wait_out(1)


_mesh = plsc.VectorSubcoreMesh(core_axis_name="c", subcore_axis_name="s")
_cparams = pltpu.CompilerParams(needs_layout_passes=False)

_phase1 = functools.partial(
    pl.kernel,
    out_type=jax.ShapeDtypeStruct((N_LINKS,), jnp.float32),
    mesh=_mesh,
    compiler_params=_cparams,
    scratch_types=(
        [pltpu.VMEM((N_NODES,), jnp.float32)]
        + [pltpu.VMEM((C1,), jnp.int32)] * 6
        + [pltpu.VMEM((C1,), jnp.float32)] * 6
        + [pltpu.SemaphoreType.DMA] * 7
    ),
)(_phase1_body)

_phase2 = functools.partial(
    pl.kernel,
    out_type=jax.ShapeDtypeStruct((N_NODES,), jnp.float32),
    mesh=_mesh,
    compiler_params=_cparams,
    scratch_types=[
        pltpu.VMEM((B2, LPN), jnp.int32),
        pltpu.VMEM((B2, LPN), jnp.int32),
        pltpu.VMEM((B2 * LPN,), jnp.int32),
        pltpu.VMEM((B2 * LPN,), jnp.int32),
        pltpu.VMEM((B2 * LPN,), jnp.float32),
        pltpu.VMEM((B2 * LPN,), jnp.float32),
        pltpu.VMEM((B2,), jnp.float32),
        pltpu.VMEM((B2,), jnp.float32),
        pltpu.SemaphoreType.DMA,
        pltpu.SemaphoreType.DMA,
        pltpu.SemaphoreType.DMA,
        pltpu.SemaphoreType.DMA,
        pltpu.SemaphoreType.DMA,
        pltpu.SemaphoreType.DMA,
    ],
)(_phase2_body)


def kernel(array, length_of_link, node_at_link_head, node_at_link_tail,
           links_at_node):
    grad = _phase1(array, length_of_link, node_at_link_head,
                   node_at_link_tail)
    return _phase2(grad, links_at_node)


# speedup vs baseline: 1.0236x; 1.0006x over previous
import functools

import jax
import jax.numpy as jnp
from jax import lax
from jax.experimental import pallas as pl
from jax.experimental.pallas import tpu as pltpu
from jax.experimental.pallas import tpu_sc as plsc

N_NODES = 100000
N_LINKS = 6400000
LPN = 64

NC = 2
NS = 16
NW = NC * NS

LINKS_PER_W = N_LINKS // NW
C1 = 2000
NCH1 = LINKS_PER_W // C1
VEC1 = C1 // 16
NBUF1 = 3

B2 = 160
NCH2 = N_NODES // B2
FULL_W = NCH2 % NW
BASE_K = NCH2 // NW


def _worker_id():
    return lax.axis_index("c") * NS + lax.axis_index("s")


def _phase1_body(array_hbm, len_hbm, head_hbm, tail_hbm, grad_hbm,
                 array_v, h0, h1, h2, t0, t1, t2,
                 l0, l1, l2, g0, g1, g2,
                 asem, ls0, ls1, ls2, ss0, ss1, ss2):
    wid = _worker_id()
    base = wid * LINKS_PER_W

    ARR_SEC = 5
    SECW = N_NODES // ARR_SEC
    for i in range(ARR_SEC):
        s = lax.rem(wid + i, ARR_SEC)
        sl = pl.ds(s * SECW, SECW)
        pltpu.async_copy(array_hbm.at[sl], array_v.at[sl], asem)

    bufs = ((h0, t0, l0, g0, ls0, ss0),
            (h1, t1, l1, g1, ls1, ss1),
            (h2, t2, l2, g2, ls2, ss2))

    def issue_loads(j, b):
        h, t, ln, _, sem, _ = bufs[b]
        off = base + j * C1
        pltpu.async_copy(head_hbm.at[pl.ds(off, C1)], h, sem)
        pltpu.async_copy(tail_hbm.at[pl.ds(off, C1)], t, sem)
        pltpu.async_copy(len_hbm.at[pl.ds(off, C1)], ln, sem)

    def wait_loads(b):
        h, t, ln, _, sem, _ = bufs[b]
        pltpu.make_async_copy(head_hbm.at[pl.ds(0, C1)], h, sem).wait()
        pltpu.make_async_copy(tail_hbm.at[pl.ds(0, C1)], t, sem).wait()
        pltpu.make_async_copy(len_hbm.at[pl.ds(0, C1)], ln, sem).wait()

    def compute(b):
        h, t, ln, g, _, _ = bufs[b]

        def inner(i, _):
            for u in range(5):
                sl = pl.ds((i * 5 + u) * 16, 16)
                ah = plsc.load_gather(array_v, [h[sl]])
                at = plsc.load_gather(array_v, [t[sl]])
                g[sl] = (ah - at) / ln[sl]
            return 0

        lax.fori_loop(0, VEC1 // 5, inner, 0)

    def issue_store(j, b):
        _, _, _, g, _, sem = bufs[b]
        pltpu.async_copy(g, grad_hbm.at[pl.ds(base + j * C1, C1)], sem)

    def wait_store(b):
        _, _, _, g, _, sem = bufs[b]
        pltpu.make_async_copy(g, grad_hbm.at[pl.ds(0, C1)], sem).wait()

    for u in range(NBUF1):
        issue_loads(u, u)
    for i in range(ARR_SEC):
        asl = pl.ds(0, SECW)
        pltpu.make_async_copy(array_hbm.at[asl], array_v.at[asl], asem).wait()

    def outer(gi, _):
        for u in range(NBUF1):
            j = gi * NBUF1 + u
            wait_loads(u)

            @pl.when(j >= NBUF1)
            def _():
                wait_store(u)

            compute(u)
            issue_store(j, u)

            @pl.when(j + NBUF1 < NCH1)
            def _():
                issue_loads(j + NBUF1, u)
        return 0

    lax.fori_loop(0, NCH1 // NBUF1, outer, 0)
    wait_loads(0)
    wait_store(0)
    compute(0)
    issue_store(NCH1 - 1, 0)
    for u in range(NBUF1):
        wait_store(u)


def _phase2_body(grad_hbm, links_hbm, out_hbm,
                 l2d0, l2d1, links0, links1, vals0, vals1, out0, out1,
                 ls0, ls1, gs0, gs1, os0, os1):
    wid = _worker_id()
    extra = wid < FULL_W
    lanes = lax.iota(jnp.int32, 16)
    MAXK = BASE_K + 1

    bufs = ((l2d0, links0, vals0, out0, ls0, gs0, os0),
            (l2d1, links1, vals1, out1, ls1, gs1, os1))

    def chunk(k):
        return wid + k * NW

    def issue_links(k, b):
        l2, _, _, _, ls, _, _ = bufs[b]
        pltpu.async_copy(
            links_hbm.at[pl.ds(chunk(k) * B2, B2), :], l2, ls)

    def wait_links(b):
        l2, _, _, _, ls, _, _ = bufs[b]
        pltpu.make_async_copy(
            links_hbm.at[pl.ds(0, B2), :], l2, ls).wait()

    def flatten(b):
        l2, l1, _, _, _, _, _ = bufs[b]

        def row(r4, _):
            for w in range(4):
                r = r4 * 4 + w
                for u in range(LPN // 16):
                    l1[pl.ds(r * LPN + u * 16, 16)] = l2[r, pl.ds(u * 16, 16)]
            return 0

        lax.fori_loop(0, B2 // 4, row, 0)

    def issue_gather(b):
        _, l1, v, _, _, gs, _ = bufs[b]
        pltpu.async_copy(grad_hbm.at[l1], v, gs)

    def wait_gather(b):
        _, l1, v, _, _, gs, _ = bufs[b]
        pltpu.make_async_copy(grad_hbm.at[l1], v, gs).wait()

    def reduce(b):
        _, _, v, o, _, _, _ = bufs[b]

        def group(g, _):
            base = (g * 16 + lanes) * LPN

            def slot8(s8, acc):
                for u in range(8):
                    acc = acc + plsc.load_gather(v, [base + (s8 * 8 + u)])
                return acc

            acc = lax.fori_loop(0, LPN // 8, slot8,
                                jnp.zeros((16,), jnp.float32))
            o[pl.ds(g * 16, 16)] = acc * (1.0 / LPN)
            return 0

        lax.fori_loop(0, B2 // 16, group, 0)

    def issue_out(k, b):
        _, _, _, o, _, _, os = bufs[b]
        pltpu.async_copy(o, out_hbm.at[pl.ds(chunk(k) * B2, B2)], os)

    def wait_out(b):
        _, _, _, o, _, _, os = bufs[b]
        pltpu.make_async_copy(o, out_hbm.at[pl.ds(0, B2)], os).wait()

    issue_links(0, 0)
    issue_links(1, 1)
    for k in range(MAXK + 1):
        b = k % 2
        ob = 1 - b
        if k < MAXK - 1:
            wait_links(b)
            flatten(b)
            issue_gather(b)
        elif k == MAXK - 1:
            @pl.when(extra)
            def _():
                wait_links(b)
                flatten(b)
                issue_gather(b)
        if k >= 1:
            def stage(ob=ob, k=k):
                wait_gather(ob)
                if k + 1 < MAXK - 1:
                    issue_links(k + 1, ob)
                elif k + 1 == MAXK - 1:
                    @pl.when(extra)
                    def _():
                        issue_links(k + 1, ob)
                if k >= 3:
                    wait_out(ob)
                reduce(ob)
                issue_out(k - 1, ob)

            if k - 1 < MAXK - 1:
                stage()
            else:
                @pl.when(extra)
                def _():
                    stage()
    wait_out(0)
    wait_out(1)


_mesh = plsc.VectorSubcoreMesh(core_axis_name="c", subcore_axis_name="s")
_cparams = pltpu.CompilerParams(needs_layout_passes=False)

_phase1 = functools.partial(
    pl.kernel,
    out_type=jax.ShapeDtypeStruct((N_LINKS,), jnp.float32),
    mesh=_mesh,
    compiler_params=_cparams,
    scratch_types=(
        [pltpu.VMEM((N_NODES,), jnp.float32)]
        + [pltpu.VMEM((C1,), jnp.int32)] * 6
        + [pltpu.VMEM((C1,), jnp.float32)] * 6
        + [pltpu.SemaphoreType.DMA] * 7
    ),
)(_phase1_body)

_phase2 = functools.partial(
    pl.kernel,
    out_type=jax.ShapeDtypeStruct((N_NODES,), jnp.float32),
    mesh=_mesh,
    compiler_params=_cparams,
    scratch_types=[
        pltpu.VMEM((B2, LPN), jnp.int32),
        pltpu.VMEM((B2, LPN), jnp.int32),
        pltpu.VMEM((B2 * LPN,), jnp.int32),
        pltpu.VMEM((B2 * LPN,), jnp.int32),
        pltpu.VMEM((B2 * LPN,), jnp.float32),
        pltpu.VMEM((B2 * LPN,), jnp.float32),
        pltpu.VMEM((B2,), jnp.float32),
        pltpu.VMEM((B2,), jnp.float32),
        pltpu.SemaphoreType.DMA,
        pltpu.SemaphoreType.DMA,
        pltpu.SemaphoreType.DMA,
        pltpu.SemaphoreType.DMA,
        pltpu.SemaphoreType.DMA,
        pltpu.SemaphoreType.DMA,
    ],
)(_phase2_body)


def kernel(array, length_of_link, node_at_link_head, node_at_link_tail,
           links_at_node):
    grad = _phase1(array, length_of_link, node_at_link_head,
                   node_at_link_tail)
    return _phase2(grad, links_at_node)


# speedup vs baseline: 1.0586x; 1.0342x over previous
import functools

import jax
import jax.numpy as jnp
from jax import lax
from jax.experimental import pallas as pl
from jax.experimental.pallas import tpu as pltpu
from jax.experimental.pallas import tpu_sc as plsc

N_NODES = 100000
N_LINKS = 6400000
LPN = 64

NC = 2
NS = 16
NW = NC * NS

LINKS_PER_W = N_LINKS // NW
C1 = 2000
NCH1 = LINKS_PER_W // C1
VEC1 = C1 // 16
NBUF1 = 3

B2 = 160
NCH2 = N_NODES // B2
FULL_W = NCH2 % NW
BASE_K = NCH2 // NW


def _worker_id():
    return lax.axis_index("c") * NS + lax.axis_index("s")


def _phase1_body(array_hbm, len_hbm, head_hbm, tail_hbm, grad_hbm,
                 array_v, arr_sh, h0, h1, h2, t0, t1, t2,
                 l0, l1, l2, g0, g1, g2,
                 ls0, ls1, ls2, ss0, ss1, ss2):
    wid = _worker_id()
    base = wid * LINKS_PER_W

    bufs = ((h0, t0, l0, g0, ls0, ss0),
            (h1, t1, l1, g1, ls1, ss1),
            (h2, t2, l2, g2, ls2, ss2))

    def issue_loads(j, b):
        h, t, ln, _, sem, _ = bufs[b]
        off = base + j * C1
        pltpu.async_copy(head_hbm.at[pl.ds(off, C1)], h, sem)
        pltpu.async_copy(tail_hbm.at[pl.ds(off, C1)], t, sem)
        pltpu.async_copy(len_hbm.at[pl.ds(off, C1)], ln, sem)

    def wait_loads(b):
        h, t, ln, _, sem, _ = bufs[b]
        pltpu.make_async_copy(head_hbm.at[pl.ds(0, C1)], h, sem).wait()
        pltpu.make_async_copy(tail_hbm.at[pl.ds(0, C1)], t, sem).wait()
        pltpu.make_async_copy(len_hbm.at[pl.ds(0, C1)], ln, sem).wait()

    def compute(b):
        h, t, ln, g, _, _ = bufs[b]

        def inner(i, _):
            for u in range(5):
                sl = pl.ds((i * 5 + u) * 16, 16)
                ah = plsc.load_gather(array_v, [h[sl]])
                at = plsc.load_gather(array_v, [t[sl]])
                g[sl] = (ah - at) / ln[sl]
            return 0

        lax.fori_loop(0, VEC1 // 5, inner, 0)

    def issue_store(j, b):
        _, _, _, g, _, sem = bufs[b]
        pltpu.async_copy(g, grad_hbm.at[pl.ds(base + j * C1, C1)], sem)

    def wait_store(b):
        _, _, _, g, _, sem = bufs[b]
        pltpu.make_async_copy(g, grad_hbm.at[pl.ds(0, C1)], sem).wait()

    for u in range(NBUF1):
        issue_loads(u, u)

    @pl.when(lax.axis_index("s") == 0)
    def _():
        pltpu.sync_copy(array_hbm, arr_sh)

    plsc.subcore_barrier()
    pltpu.sync_copy(arr_sh, array_v)

    def outer(gi, _):
        for u in range(NBUF1):
            j = gi * NBUF1 + u
            wait_loads(u)

            @pl.when(j >= NBUF1)
            def _():
                wait_store(u)

            compute(u)
            issue_store(j, u)

            @pl.when(j + NBUF1 < NCH1)
            def _():
                issue_loads(j + NBUF1, u)
        return 0

    lax.fori_loop(0, NCH1 // NBUF1, outer, 0)
    wait_loads(0)
    wait_store(0)
    compute(0)
    issue_store(NCH1 - 1, 0)
    for u in range(NBUF1):
        wait_store(u)


def _phase2_body(grad_hbm, links_hbm, out_hbm,
                 l2d0, l2d1, links0, links1, vals0, vals1, out0, out1,
                 ls0, ls1, gs0, gs1, os0, os1):
    wid = _worker_id()
    extra = wid < FULL_W
    lanes = lax.iota(jnp.int32, 16)
    MAXK = BASE_K + 1

    bufs = ((l2d0, links0, vals0, out0, ls0, gs0, os0),
            (l2d1, links1, vals1, out1, ls1, gs1, os1))

    def chunk(k):
        return wid + k * NW

    def issue_links(k, b):
        l2, _, _, _, ls, _, _ = bufs[b]
        pltpu.async_copy(
            links_hbm.at[pl.ds(chunk(k) * B2, B2), :], l2, ls)

    def wait_links(b):
        l2, _, _, _, ls, _, _ = bufs[b]
        pltpu.make_async_copy(
            links_hbm.at[pl.ds(0, B2), :], l2, ls).wait()

    def flatten(b):
        l2, l1, _, _, _, _, _ = bufs[b]

        def row(r4, _):
            for w in range(4):
                r = r4 * 4 + w
                for u in range(LPN // 16):
                    l1[pl.ds(r * LPN + u * 16, 16)] = l2[r, pl.ds(u * 16, 16)]
            return 0

        lax.fori_loop(0, B2 // 4, row, 0)

    def issue_gather(b):
        _, l1, v, _, _, gs, _ = bufs[b]
        pltpu.async_copy(grad_hbm.at[l1], v, gs)

    def wait_gather(b):
        _, l1, v, _, _, gs, _ = bufs[b]
        pltpu.make_async_copy(grad_hbm.at[l1], v, gs).wait()

    def reduce(b):
        _, _, v, o, _, _, _ = bufs[b]

        def group(g, _):
            base = (g * 16 + lanes) * LPN

            def slot8(s8, acc):
                for u in range(8):
                    acc = acc + plsc.load_gather(v, [base + (s8 * 8 + u)])
                return acc

            acc = lax.fori_loop(0, LPN // 8, slot8,
                                jnp.zeros((16,), jnp.float32))
            o[pl.ds(g * 16, 16)] = acc * (1.0 / LPN)
            return 0

        lax.fori_loop(0, B2 // 16, group, 0)

    def issue_out(k, b):
        _, _, _, o, _, _, os = bufs[b]
        pltpu.async_copy(o, out_hbm.at[pl.ds(chunk(k) * B2, B2)], os)

    def wait_out(b):
        _, _, _, o, _, _, os = bufs[b]
        pltpu.make_async_copy(o, out_hbm.at[pl.ds(0, B2)], os).wait()

    issue_links(0, 0)
    issue_links(1, 1)
    for k in range(MAXK + 1):
        b = k % 2
        ob = 1 - b
        if k < MAXK - 1:
            wait_links(b)
            flatten(b)
            issue_gather(b)
        elif k == MAXK - 1:
            @pl.when(extra)
            def _():
                wait_links(b)
                flatten(b)
                issue_gather(b)
        if k >= 1:
            def stage(ob=ob, k=k):
                wait_gather(ob)
                if k + 1 < MAXK - 1:
                    issue_links(k + 1, ob)
                elif k + 1 == MAXK - 1:
                    @pl.when(extra)
                    def _():
                        issue_links(k + 1, ob)
                if k >= 3:
                    wait_out(ob)
                reduce(ob)
                issue_out(k - 1, ob)

            if k - 1 < MAXK - 1:
                stage()
            else:
                @pl.when(extra)
                def _():
                    stage()
    wait_out(0)
    wait_out(1)


_mesh = plsc.VectorSubcoreMesh(core_axis_name="c", subcore_axis_name="s")
_cparams = pltpu.CompilerParams(needs_layout_passes=False)

_phase1 = functools.partial(
    pl.kernel,
    out_type=jax.ShapeDtypeStruct((N_LINKS,), jnp.float32),
    mesh=_mesh,
    compiler_params=_cparams,
    scratch_types=(
        [pltpu.VMEM((N_NODES,), jnp.float32),
         pltpu.VMEM_SHARED((N_NODES,), jnp.float32)]
        + [pltpu.VMEM((C1,), jnp.int32)] * 6
        + [pltpu.VMEM((C1,), jnp.float32)] * 6
        + [pltpu.SemaphoreType.DMA] * 6
    ),
)(_phase1_body)

_phase2 = functools.partial(
    pl.kernel,
    out_type=jax.ShapeDtypeStruct((N_NODES,), jnp.float32),
    mesh=_mesh,
    compiler_params=_cparams,
    scratch_types=[
        pltpu.VMEM((B2, LPN), jnp.int32),
        pltpu.VMEM((B2, LPN), jnp.int32),
        pltpu.VMEM((B2 * LPN,), jnp.int32),
        pltpu.VMEM((B2 * LPN,), jnp.int32),
        pltpu.VMEM((B2 * LPN,), jnp.float32),
        pltpu.VMEM((B2 * LPN,), jnp.float32),
        pltpu.VMEM((B2,), jnp.float32),
        pltpu.VMEM((B2,), jnp.float32),
        pltpu.SemaphoreType.DMA,
        pltpu.SemaphoreType.DMA,
        pltpu.SemaphoreType.DMA,
        pltpu.SemaphoreType.DMA,
        pltpu.SemaphoreType.DMA,
        pltpu.SemaphoreType.DMA,
    ],
)(_phase2_body)


def kernel(array, length_of_link, node_at_link_head, node_at_link_tail,
           links_at_node):
    grad = _phase1(array, length_of_link, node_at_link_head,
                   node_at_link_tail)
    return _phase2(grad, links_at_node)
